# Initial kernel scaffold; baseline (speedup 1.0000x reference)
import jax
import jax.numpy as jnp
from jax.experimental import pallas as pl


def kernel(x, edge_index, edge_attr, batch_index, Wmsg1, bmsg1, Wedge1, bedge1, att1, Wself1, bself1, Wmsg2, bmsg2, Wedge2, bedge2, att2, Wself2, bself2, Wmsg3, bmsg3, Wedge3, bedge3, att3, Wself3, bself3, Wout, bout):
    raise NotImplementedError("write your pallas kernel here")


# speedup vs baseline: 42.8425x; 42.8425x over previous
import functools

import jax
import jax.numpy as jnp
from jax import lax
from jax.experimental import pallas as pl
from jax.experimental.pallas import tpu as pltpu
from jax.experimental.pallas import tpu_sc as plsc

_N = 10000
_E = 320000
_H = 5
_FIN = 128
_NG = 64
_NC = 2
_NS = 16
_NW = _NC * _NS
_EPW = _E // _NW
_CH = 80
_NCHUNK = _EPW // _CH
_GR = _CH // 16
_BN = 1000
_GN = _N // _BN
_BE = 32000
_NA = 10240

_LAYERS = {1: (4, 48, 48), 2: (8, 64, 64), 3: (16, 96, 96)}


def _full16(v):
    return jnp.full((16,), v, jnp.int32)


def _make_sc_edge(O, K_in, K_out):
    HO = _H * O
    rpt = _NA // _NS
    mesh = plsc.VectorSubcoreMesh(core_axis_name="c", subcore_axis_name="s")

    @functools.partial(
        pl.kernel, mesh=mesh,
        compiler_params=pltpu.CompilerParams(
            use_tc_tiling_on_sc=False, needs_layout_passes=False),
        out_type=jax.ShapeDtypeStruct((_NC, _NA, K_out), jnp.float32),
        scratch_types=[
            pltpu.VMEM((_NCHUNK, _CH), jnp.int32),
            pltpu.VMEM((_NCHUNK, _CH), jnp.int32),
            pltpu.VMEM((_EPW,), jnp.float32),
            pltpu.VMEM((8, 16), jnp.float32),
            pltpu.VMEM((_CH, K_in), jnp.float32),
            pltpu.VMEM((_CH, K_out), jnp.float32),
            pltpu.VMEM((_CH * 16,), jnp.float32),
            pltpu.VMEM_SHARED((_NA, K_out), jnp.float32),
            pltpu.SemaphoreType.DMA,
        ],
    )
    def k(src_hbm, dst_hbm, attr_hbm, consts_hbm, xmtab_hbm, zeros_hbm,
          out_hbm, src_v, dst_v, attr_v, consts_v, in_v, out_v, eabuf,
          accum, sem):
        c = lax.axis_index("c")
        s = lax.axis_index("s")
        wid = s * _NC + c
        pltpu.sync_copy(src_hbm.at[wid], src_v)
        pltpu.sync_copy(dst_hbm.at[wid], dst_v)
        pltpu.sync_copy(attr_hbm.at[wid], attr_v)
        pltpu.sync_copy(consts_hbm, consts_v)
        rpt = _NA // _NS
        r0 = s * rpt
        pltpu.sync_copy(zeros_hbm.at[pl.ds(r0, rpt)],
                        accum.at[pl.ds(r0, rpt)])
        plsc.subcore_barrier()

        iota = lax.iota(jnp.int32, 16)
        cev = consts_v[0]
        dev = consts_v[1]
        Cv = consts_v[2]
        idxc = jnp.minimum(jnp.where(iota < 8, iota, iota - 8), 5)
        nblk = (K_in - 16) // 16
        hidx = [jnp.minimum((b * 16 + iota) // O, 15) for b in range(nblk)]
        ones = jnp.full((16,), 1.0, jnp.float32)

        def chunk(j, carry):
            pltpu.async_copy(xmtab_hbm.at[src_v.at[j]], in_v, sem).wait()
            for e in range(_CH):
                arow = in_v[e, pl.ds(0, 16)]
                attrb = plsc.load_gather(
                    attr_v, [jnp.full((16,), j * _CH + e, jnp.int32)])
                al = arow + attrb * cev + dev
                al = jnp.maximum(al, 0.2 * al)
                ea = jnp.exp(al - Cv)
                eabuf[pl.ds(e * 16, 16)] = ea
                f16 = plsc.load_gather(eabuf, [idxc + (e * 16)])
                sel = jnp.where(iota < 8, ones, attrb)
                out_v[e, pl.ds(0, 16)] = f16 * sel
                for b in range(nblk):
                    xv = in_v[e, pl.ds(16 + b * 16, 16)]
                    eab = plsc.load_gather(eabuf, [hidx[b] + (e * 16)])
                    out_v[e, pl.ds(16 + b * 16, 16)] = eab * xv
            pltpu.sync_copy(out_v, accum.at[dst_v.at[j]], add=True)
            return carry

        lax.fori_loop(0, _NCHUNK, chunk, 0)
        plsc.subcore_barrier()
        pltpu.sync_copy(accum.at[pl.ds(r0, rpt)],
                        out_hbm.at[c, pl.ds(r0, rpt)])

    return k


def _project(xb, Wm, bm, attR, Ws, bs, O, K_in):
    B = xb.shape[0]
    xm = jnp.dot(xb, Wm, precision=lax.Precision.HIGHEST,
                 preferred_element_type=jnp.float32) + bm
    an = (xm.reshape(B, _H, O) * attR[None]).sum(-1)
    xself = jnp.dot(xb, Ws, precision=lax.Precision.HIGHEST,
                    preferred_element_type=jnp.float32) + bs
    parts = [an, jnp.zeros((B, 11), jnp.float32), xm]
    pad = K_in - 16 - _H * O
    if pad:
        parts.append(jnp.zeros((B, pad), jnp.float32))
    return jnp.concatenate(parts, axis=1), xself, an.max(0)


def _anmax8(anmax):
    return jnp.broadcast_to(
        jnp.concatenate([anmax, jnp.full((3,), -1e30, jnp.float32)])[None, :],
        (8, 8))


def _tc_project1(O, K_in):
    def body(x_ref, attr_ref, Wm_ref, bm_ref, attR_ref, Ws_ref, bs_ref,
             xmtab_ref, xself_ref, anmax_ref, attmm_ref):
        i = pl.program_id(0)
        xmtab, xself, anmax = _project(
            x_ref[...], Wm_ref[...], bm_ref[...], attR_ref[...],
            Ws_ref[...], bs_ref[...], O, K_in)
        xmtab_ref[...] = xmtab
        xself_ref[...] = xself
        am = _anmax8(anmax)
        ab = attr_ref[0]
        amm = jnp.concatenate(
            [jnp.full((8, 4), jnp.max(ab), jnp.float32),
             jnp.full((8, 4), jnp.max(-ab), jnp.float32)], axis=1)

        @pl.when(i == 0)
        def _():
            anmax_ref[...] = am
            attmm_ref[...] = amm

        @pl.when(i > 0)
        def _():
            anmax_ref[...] = jnp.maximum(anmax_ref[...], am)
            attmm_ref[...] = jnp.maximum(attmm_ref[...], amm)

    full = lambda i: (0, 0)
    return pl.pallas_call(
        body,
        grid=(_GN,),
        in_specs=[
            pl.BlockSpec((_BN, _FIN), lambda i: (i, 0)),
            pl.BlockSpec((1, 1, _BE), lambda i: (i, 0, 0)),
            pl.BlockSpec((_FIN, _H * O), full),
            pl.BlockSpec((1, _H * O), full),
            pl.BlockSpec((_H, O), full),
            pl.BlockSpec((_FIN, O), full),
            pl.BlockSpec((1, O), full),
        ],
        out_specs=[
            pl.BlockSpec((_BN, K_in), lambda i: (i, 0)),
            pl.BlockSpec((_BN, O), lambda i: (i, 0)),
            pl.BlockSpec((8, 8), full),
            pl.BlockSpec((8, 8), full),
        ],
        out_shape=[
            jax.ShapeDtypeStruct((_N, K_in), jnp.float32),
            jax.ShapeDtypeStruct((_N, O), jnp.float32),
            jax.ShapeDtypeStruct((8, 8), jnp.float32),
            jax.ShapeDtypeStruct((8, 8), jnp.float32),
        ],
    )


def _finalize_block(P, xself, WeR, beR, Op):
    S = P[0] + P[1]
    S1 = S[:, 0:_H]
    S2 = S[:, 8:8 + _H]
    S3 = S[:, 16:16 + _H * Op].reshape(-1, _H, Op)
    num = S3 + WeR[None] * S2[:, :, None] + beR[None] * S1[:, :, None]
    return jax.nn.relu(xself + (num / (S1[:, :, None] + 1e-16)).mean(1))


def _tc_finalize_project(Op, Kp, O, K_in):
    def body(P_ref, xselfp_ref, WeR_ref, beR_ref,
             Wm_ref, bm_ref, attR_ref, Ws_ref, bs_ref,
             xmtab_ref, xself_ref, anmax_ref):
        i = pl.program_id(0)
        xn = _finalize_block(P_ref[...], xselfp_ref[...], WeR_ref[...],
                             beR_ref[...], Op)
        xmtab, xself, anmax = _project(
            xn, Wm_ref[...], bm_ref[...], attR_ref[...],
            Ws_ref[...], bs_ref[...], O, K_in)
        xmtab_ref[...] = xmtab
        xself_ref[...] = xself
        am = _anmax8(anmax)

        @pl.when(i == 0)
        def _():
            anmax_ref[...] = am

        @pl.when(i > 0)
        def _():
            anmax_ref[...] = jnp.maximum(anmax_ref[...], am)

    full = lambda i: (0, 0)
    return pl.pallas_call(
        body,
        grid=(_GN,),
        in_specs=[
            pl.BlockSpec((_NC, _BN, Kp), lambda i: (0, i, 0)),
            pl.BlockSpec((_BN, Op), lambda i: (i, 0)),
            pl.BlockSpec((_H, Op), full),
            pl.BlockSpec((_H, Op), full),
            pl.BlockSpec((Op, _H * O), full),
            pl.BlockSpec((1, _H * O), full),
            pl.BlockSpec((_H, O), full),
            pl.BlockSpec((Op, O), full),
            pl.BlockSpec((1, O), full),
        ],
        out_specs=[
            pl.BlockSpec((_BN, K_in), lambda i: (i, 0)),
            pl.BlockSpec((_BN, O), lambda i: (i, 0)),
            pl.BlockSpec((8, 8), full),
        ],
        out_shape=[
            jax.ShapeDtypeStruct((_N, K_in), jnp.float32),
            jax.ShapeDtypeStruct((_N, O), jnp.float32),
            jax.ShapeDtypeStruct((8, 8), jnp.float32),
        ],
    )


def _tc_final_pool(Op, Kp):
    def body(P_ref, xselfp_ref, WeR_ref, beR_ref, batch_ref,
             Wout_ref, bout_ref, out_ref, gp_ref):
        i = pl.program_id(0)
        x3 = _finalize_block(P_ref[...], xselfp_ref[...], WeR_ref[...],
                             beR_ref[...], Op)
        b = batch_ref[0]
        gids = lax.broadcasted_iota(jnp.int32, (_NG, _BN), 0)
        mf = (b == gids).astype(jnp.float32)
        bmax = (mf[:, :, None] * (x3[None, :, :] + 1.0) - 1.0).max(1)

        @pl.when(i == 0)
        def _():
            gp_ref[...] = bmax

        @pl.when(i > 0)
        def _():
            gp_ref[...] = jnp.maximum(gp_ref[...], bmax)

        @pl.when(i == _GN - 1)
        def _():
            g2 = jnp.where(gp_ref[...] >= 0.0, gp_ref[...], 0.0)
            out_ref[...] = (jnp.dot(g2, Wout_ref[...],
                                    precision=lax.Precision.HIGHEST,
                                    preferred_element_type=jnp.float32)
                            + bout_ref[...])

    full = lambda i: (0, 0)
    return pl.pallas_call(
        body,
        grid=(_GN,),
        in_specs=[
            pl.BlockSpec((_NC, _BN, Kp), lambda i: (0, i, 0)),
            pl.BlockSpec((_BN, Op), lambda i: (i, 0)),
            pl.BlockSpec((_H, Op), full),
            pl.BlockSpec((_H, Op), full),
            pl.BlockSpec((1, 1, _BN), lambda i: (i, 0, 0)),
            pl.BlockSpec((Op, 2), full),
            pl.BlockSpec((1, 2), full),
        ],
        out_specs=pl.BlockSpec((_NG, 2), full),
        out_shape=jax.ShapeDtypeStruct((_NG, 2), jnp.float32),
        scratch_shapes=[pltpu.VMEM((_NG, Op), jnp.float32)],
    )


def _head_consts(WeR, beR, attR, anmax8, attmm):
    ce = (WeR * attR).sum(-1)
    de = (beR * attR).sum(-1)
    amax = attmm[0, 0]
    amin = -attmm[0, 4]
    M = anmax8[0, :_H] + jnp.maximum(amax * ce, amin * ce) + de
    C = jnp.maximum(M, 0.2 * M)
    z11 = jnp.zeros((11,), jnp.float32)
    rows = [jnp.concatenate([v, z11])[None, :] for v in (ce, de, C)]
    return jnp.concatenate(rows + [jnp.zeros((5, 16), jnp.float32)], axis=0)


def kernel(x, edge_index, edge_attr, batch_index,
           Wmsg1, bmsg1, Wedge1, bedge1, att1, Wself1, bself1,
           Wmsg2, bmsg2, Wedge2, bedge2, att2, Wself2, bself2,
           Wmsg3, bmsg3, Wedge3, bedge3, att3, Wself3, bself3,
           Wout, bout):
    src = edge_index[0].reshape(_NW, _NCHUNK, _CH)
    dst = edge_index[1].reshape(_NW, _NCHUNK, _CH)
    attr_flat = edge_attr.reshape(_E)
    attrw = attr_flat.reshape(_NW, _EPW)
    attr2d = attr_flat.reshape(_GN, 1, _BE)
    zeros = {k: jnp.zeros((_NA, ko), jnp.float32)
             for k, (_, _, ko) in _LAYERS.items()}
    batch3d = batch_index.reshape(_GN, 1, _BN)

    Ws = {
        1: (Wmsg1, bmsg1.reshape(1, -1), Wedge1.reshape(_H, -1),
            bedge1.reshape(_H, -1), att1[0], Wself1, bself1.reshape(1, -1)),
        2: (Wmsg2, bmsg2.reshape(1, -1), Wedge2.reshape(_H, -1),
            bedge2.reshape(_H, -1), att2[0], Wself2, bself2.reshape(1, -1)),
        3: (Wmsg3, bmsg3.reshape(1, -1), Wedge3.reshape(_H, -1),
            bedge3.reshape(_H, -1), att3[0], Wself3, bself3.reshape(1, -1)),
    }

    O1, Kin1, Kout1 = _LAYERS[1]
    Wm, bm, WeR1, beR1, attR1, Wsf, bsf = Ws[1]
    xmtab1, xself1, anmax1, attmm = _tc_project1(O1, Kin1)(
        x, attr2d, Wm, bm, attR1, Wsf, bsf)
    consts1 = _head_consts(WeR1, beR1, attR1, anmax1, attmm)

    P1 = _make_sc_edge(O1, Kin1, Kout1)(
        src, dst, attrw, consts1, xmtab1, zeros[1])

    O2, Kin2, Kout2 = _LAYERS[2]
    Wm, bm, WeR2, beR2, attR2, Wsf, bsf = Ws[2]
    xmtab2, xself2, anmax2 = _tc_finalize_project(O1, Kout1, O2, Kin2)(
        P1, xself1, WeR1, beR1, Wm, bm, attR2, Wsf, bsf)
    consts2 = _head_consts(WeR2, beR2, attR2, anmax2, attmm)
    P2 = _make_sc_edge(O2, Kin2, Kout2)(
        src, dst, attrw, consts2, xmtab2, zeros[2])

    O3, Kin3, Kout3 = _LAYERS[3]
    Wm, bm, WeR3, beR3, attR3, Wsf, bsf = Ws[3]
    xmtab3, xself3, anmax3 = _tc_finalize_project(O2, Kout2, O3, Kin3)(
        P2, xself2, WeR2, beR2, Wm, bm, attR3, Wsf, bsf)
    consts3 = _head_consts(WeR3, beR3, attR3, anmax3, attmm)
    P3 = _make_sc_edge(O3, Kin3, Kout3)(
        src, dst, attrw, consts3, xmtab3, zeros[3])

    return _tc_final_pool(O3, Kout3)(
        P3, xself3, WeR3, beR3, batch3d, Wout, bout.reshape(1, 2))


# speedup vs baseline: 49.3749x; 1.1525x over previous
import functools

import jax
import jax.numpy as jnp
from jax import lax
from jax.experimental import pallas as pl
from jax.experimental.pallas import tpu as pltpu
from jax.experimental.pallas import tpu_sc as plsc

_N = 10000
_E = 320000
_H = 5
_FIN = 128
_NG = 64
_NC = 2
_NS = 16
_NW = _NC * _NS
_EPW = _E // _NW
_CH = 80
_NCHUNK = _EPW // _CH
_GR = _CH // 16
_BN = 1000
_GN = _N // _BN
_BE = 32000
_NA = 10240

_LAYERS = {1: (4, 48, 64), 2: (8, 64, 80), 3: (16, 96, 112)}


def _full16(v):
    return jnp.full((16,), v, jnp.int32)


def _make_sc_edge(O, K_in, K_out):
    HO = _H * O
    rpt = _NA // _NS
    mesh = plsc.VectorSubcoreMesh(core_axis_name="c", subcore_axis_name="s")

    @functools.partial(
        pl.kernel, mesh=mesh,
        compiler_params=pltpu.CompilerParams(
            use_tc_tiling_on_sc=False, needs_layout_passes=False),
        out_type=jax.ShapeDtypeStruct((_NC, _NA, K_out), jnp.float32),
        scratch_types=[
            pltpu.VMEM((_NCHUNK, _CH), jnp.int32),
            pltpu.VMEM((_NCHUNK, _CH), jnp.int32),
            pltpu.VMEM((_EPW,), jnp.float32),
            pltpu.VMEM((8, 16), jnp.float32),
            pltpu.VMEM((2, _CH, K_in), jnp.float32),
            pltpu.VMEM((_CH, K_out), jnp.float32),
            pltpu.VMEM((_CH * 16,), jnp.float32),
            pltpu.VMEM_SHARED((_NA, K_out), jnp.float32),
            pltpu.SemaphoreType.DMA((2,)),
        ],
    )
    def k(src_hbm, dst_hbm, attr_hbm, consts_hbm, xmtab_hbm, zeros_hbm,
          out_hbm, src_v, dst_v, attr_v, consts_v, in_d, out_v, eabuf,
          accum, gsem):
        c = lax.axis_index("c")
        s = lax.axis_index("s")
        wid = s * _NC + c
        pltpu.sync_copy(src_hbm.at[wid], src_v)
        pltpu.sync_copy(dst_hbm.at[wid], dst_v)
        pltpu.sync_copy(attr_hbm.at[wid], attr_v)
        pltpu.sync_copy(consts_hbm, consts_v)
        rpt = _NA // _NS
        r0 = s * rpt
        pltpu.sync_copy(zeros_hbm.at[pl.ds(r0, rpt)],
                        accum.at[pl.ds(r0, rpt)])
        plsc.subcore_barrier()

        iota = lax.iota(jnp.int32, 16)
        cev = consts_v[0]
        dev = consts_v[1]
        Cv = consts_v[2]
        nblk = (K_in - 16) // 16
        hidx = [jnp.minimum((b * 16 + iota) // O, 15) for b in range(nblk)]

        pltpu.async_copy(xmtab_hbm.at[src_v.at[0]], in_d.at[0], gsem.at[0])

        def chunk(j, carry):
            p = jnp.bitwise_and(j, 1)
            pn = 1 - p

            @pl.when(j < _NCHUNK - 1)
            def _():
                pltpu.async_copy(xmtab_hbm.at[src_v.at[j + 1]],
                                 in_d.at[pn], gsem.at[pn])

            pltpu.make_async_copy(xmtab_hbm.at[src_v.at[j]],
                                  in_d.at[p], gsem.at[p]).wait()
            for e in range(_CH):
                arow = in_d[p, e, pl.ds(0, 16)]
                attrb = plsc.load_gather(
                    attr_v, [jnp.full((16,), j * _CH + e, jnp.int32)])
                al = arow + attrb * cev + dev
                al = jnp.maximum(al, 0.2 * al)
                ea = jnp.exp(al - Cv)
                eabuf[pl.ds(e * 16, 16)] = ea
                out_v[e, pl.ds(0, 16)] = ea
                out_v[e, pl.ds(16, 16)] = ea * attrb
            for e in range(_CH):
                for b in range(nblk):
                    xv = in_d[p, e, pl.ds(16 + b * 16, 16)]
                    eab = plsc.load_gather(eabuf, [hidx[b] + (e * 16)])
                    out_v[e, pl.ds(32 + b * 16, 16)] = eab * xv
            pltpu.sync_copy(out_v, accum.at[dst_v.at[j]], add=True)
            return carry

        lax.fori_loop(0, _NCHUNK, chunk, 0)
        plsc.subcore_barrier()
        pltpu.sync_copy(accum.at[pl.ds(r0, rpt)],
                        out_hbm.at[c, pl.ds(r0, rpt)])

    return k


def _project(xb, Wm, bm, attR, Ws, bs, O, K_in):
    B = xb.shape[0]
    xm = jnp.dot(xb, Wm, precision=lax.Precision.HIGHEST,
                 preferred_element_type=jnp.float32) + bm
    an = (xm.reshape(B, _H, O) * attR[None]).sum(-1)
    xself = jnp.dot(xb, Ws, precision=lax.Precision.HIGHEST,
                    preferred_element_type=jnp.float32) + bs
    parts = [an, jnp.zeros((B, 11), jnp.float32), xm]
    pad = K_in - 16 - _H * O
    if pad:
        parts.append(jnp.zeros((B, pad), jnp.float32))
    return jnp.concatenate(parts, axis=1), xself, an.max(0)


def _anmax8(anmax):
    return jnp.broadcast_to(
        jnp.concatenate([anmax, jnp.full((3,), -1e30, jnp.float32)])[None, :],
        (8, 8))


def _tc_project1(O, K_in):
    def body(x_ref, attr_ref, Wm_ref, bm_ref, attR_ref, Ws_ref, bs_ref,
             xmtab_ref, xself_ref, anmax_ref, attmm_ref):
        i = pl.program_id(0)
        xmtab, xself, anmax = _project(
            x_ref[...], Wm_ref[...], bm_ref[...], attR_ref[...],
            Ws_ref[...], bs_ref[...], O, K_in)
        xmtab_ref[...] = xmtab
        xself_ref[...] = xself
        am = _anmax8(anmax)
        ab = attr_ref[0]
        amm = jnp.concatenate(
            [jnp.full((8, 4), jnp.max(ab), jnp.float32),
             jnp.full((8, 4), jnp.max(-ab), jnp.float32)], axis=1)

        @pl.when(i == 0)
        def _():
            anmax_ref[...] = am
            attmm_ref[...] = amm

        @pl.when(i > 0)
        def _():
            anmax_ref[...] = jnp.maximum(anmax_ref[...], am)
            attmm_ref[...] = jnp.maximum(attmm_ref[...], amm)

    full = lambda i: (0, 0)
    return pl.pallas_call(
        body,
        grid=(_GN,),
        in_specs=[
            pl.BlockSpec((_BN, _FIN), lambda i: (i, 0)),
            pl.BlockSpec((1, 1, _BE), lambda i: (i, 0, 0)),
            pl.BlockSpec((_FIN, _H * O), full),
            pl.BlockSpec((1, _H * O), full),
            pl.BlockSpec((_H, O), full),
            pl.BlockSpec((_FIN, O), full),
            pl.BlockSpec((1, O), full),
        ],
        out_specs=[
            pl.BlockSpec((_BN, K_in), lambda i: (i, 0)),
            pl.BlockSpec((_BN, O), lambda i: (i, 0)),
            pl.BlockSpec((8, 8), full),
            pl.BlockSpec((8, 8), full),
        ],
        out_shape=[
            jax.ShapeDtypeStruct((_N, K_in), jnp.float32),
            jax.ShapeDtypeStruct((_N, O), jnp.float32),
            jax.ShapeDtypeStruct((8, 8), jnp.float32),
            jax.ShapeDtypeStruct((8, 8), jnp.float32),
        ],
    )


def _finalize_block(P, xself, WeR, beR, Op):
    S = P[0] + P[1]
    S1 = S[:, 0:_H]
    S2 = S[:, 16:16 + _H]
    S3 = S[:, 32:32 + _H * Op].reshape(-1, _H, Op)
    num = S3 + WeR[None] * S2[:, :, None] + beR[None] * S1[:, :, None]
    return jax.nn.relu(xself + (num / (S1[:, :, None] + 1e-16)).mean(1))


def _tc_finalize_project(Op, Kp, O, K_in):
    def body(P_ref, xselfp_ref, WeR_ref, beR_ref,
             Wm_ref, bm_ref, attR_ref, Ws_ref, bs_ref,
             xmtab_ref, xself_ref, anmax_ref):
        i = pl.program_id(0)
        xn = _finalize_block(P_ref[...], xselfp_ref[...], WeR_ref[...],
                             beR_ref[...], Op)
        xmtab, xself, anmax = _project(
            xn, Wm_ref[...], bm_ref[...], attR_ref[...],
            Ws_ref[...], bs_ref[...], O, K_in)
        xmtab_ref[...] = xmtab
        xself_ref[...] = xself
        am = _anmax8(anmax)

        @pl.when(i == 0)
        def _():
            anmax_ref[...] = am

        @pl.when(i > 0)
        def _():
            anmax_ref[...] = jnp.maximum(anmax_ref[...], am)

    full = lambda i: (0, 0)
    return pl.pallas_call(
        body,
        grid=(_GN,),
        in_specs=[
            pl.BlockSpec((_NC, _BN, Kp), lambda i: (0, i, 0)),
            pl.BlockSpec((_BN, Op), lambda i: (i, 0)),
            pl.BlockSpec((_H, Op), full),
            pl.BlockSpec((_H, Op), full),
            pl.BlockSpec((Op, _H * O), full),
            pl.BlockSpec((1, _H * O), full),
            pl.BlockSpec((_H, O), full),
            pl.BlockSpec((Op, O), full),
            pl.BlockSpec((1, O), full),
        ],
        out_specs=[
            pl.BlockSpec((_BN, K_in), lambda i: (i, 0)),
            pl.BlockSpec((_BN, O), lambda i: (i, 0)),
            pl.BlockSpec((8, 8), full),
        ],
        out_shape=[
            jax.ShapeDtypeStruct((_N, K_in), jnp.float32),
            jax.ShapeDtypeStruct((_N, O), jnp.float32),
            jax.ShapeDtypeStruct((8, 8), jnp.float32),
        ],
    )


def _tc_final_pool(Op, Kp):
    def body(P_ref, xselfp_ref, WeR_ref, beR_ref, batch_ref,
             Wout_ref, bout_ref, out_ref, gp_ref):
        i = pl.program_id(0)
        x3 = _finalize_block(P_ref[...], xselfp_ref[...], WeR_ref[...],
                             beR_ref[...], Op)
        b = batch_ref[0]
        gids = lax.broadcasted_iota(jnp.int32, (_NG, _BN), 0)
        mf = (b == gids).astype(jnp.float32)
        bmax = (mf[:, :, None] * (x3[None, :, :] + 1.0) - 1.0).max(1)

        @pl.when(i == 0)
        def _():
            gp_ref[...] = bmax

        @pl.when(i > 0)
        def _():
            gp_ref[...] = jnp.maximum(gp_ref[...], bmax)

        @pl.when(i == _GN - 1)
        def _():
            g2 = jnp.where(gp_ref[...] >= 0.0, gp_ref[...], 0.0)
            out_ref[...] = (jnp.dot(g2, Wout_ref[...],
                                    precision=lax.Precision.HIGHEST,
                                    preferred_element_type=jnp.float32)
                            + bout_ref[...])

    full = lambda i: (0, 0)
    return pl.pallas_call(
        body,
        grid=(_GN,),
        in_specs=[
            pl.BlockSpec((_NC, _BN, Kp), lambda i: (0, i, 0)),
            pl.BlockSpec((_BN, Op), lambda i: (i, 0)),
            pl.BlockSpec((_H, Op), full),
            pl.BlockSpec((_H, Op), full),
            pl.BlockSpec((1, 1, _BN), lambda i: (i, 0, 0)),
            pl.BlockSpec((Op, 2), full),
            pl.BlockSpec((1, 2), full),
        ],
        out_specs=pl.BlockSpec((_NG, 2), full),
        out_shape=jax.ShapeDtypeStruct((_NG, 2), jnp.float32),
        scratch_shapes=[pltpu.VMEM((_NG, Op), jnp.float32)],
    )


def _head_consts(WeR, beR, attR, anmax8, attmm):
    ce = (WeR * attR).sum(-1)
    de = (beR * attR).sum(-1)
    amax = attmm[0, 0]
    amin = -attmm[0, 4]
    M = anmax8[0, :_H] + jnp.maximum(amax * ce, amin * ce) + de
    C = jnp.maximum(M, 0.2 * M)
    z11 = jnp.zeros((11,), jnp.float32)
    rows = [jnp.concatenate([v, z11])[None, :] for v in (ce, de, C)]
    return jnp.concatenate(rows + [jnp.zeros((5, 16), jnp.float32)], axis=0)


def kernel(x, edge_index, edge_attr, batch_index,
           Wmsg1, bmsg1, Wedge1, bedge1, att1, Wself1, bself1,
           Wmsg2, bmsg2, Wedge2, bedge2, att2, Wself2, bself2,
           Wmsg3, bmsg3, Wedge3, bedge3, att3, Wself3, bself3,
           Wout, bout):
    src = edge_index[0].reshape(_NW, _NCHUNK, _CH)
    dst = edge_index[1].reshape(_NW, _NCHUNK, _CH)
    attr_flat = edge_attr.reshape(_E)
    attrw = attr_flat.reshape(_NW, _EPW)
    attr2d = attr_flat.reshape(_GN, 1, _BE)
    zeros = {k: jnp.zeros((_NA, ko), jnp.float32)
             for k, (_, _, ko) in _LAYERS.items()}
    batch3d = batch_index.reshape(_GN, 1, _BN)

    Ws = {
        1: (Wmsg1, bmsg1.reshape(1, -1), Wedge1.reshape(_H, -1),
            bedge1.reshape(_H, -1), att1[0], Wself1, bself1.reshape(1, -1)),
        2: (Wmsg2, bmsg2.reshape(1, -1), Wedge2.reshape(_H, -1),
            bedge2.reshape(_H, -1), att2[0], Wself2, bself2.reshape(1, -1)),
        3: (Wmsg3, bmsg3.reshape(1, -1), Wedge3.reshape(_H, -1),
            bedge3.reshape(_H, -1), att3[0], Wself3, bself3.reshape(1, -1)),
    }

    O1, Kin1, Kout1 = _LAYERS[1]
    Wm, bm, WeR1, beR1, attR1, Wsf, bsf = Ws[1]
    xmtab1, xself1, anmax1, attmm = _tc_project1(O1, Kin1)(
        x, attr2d, Wm, bm, attR1, Wsf, bsf)
    consts1 = _head_consts(WeR1, beR1, attR1, anmax1, attmm)

    P1 = _make_sc_edge(O1, Kin1, Kout1)(
        src, dst, attrw, consts1, xmtab1, zeros[1])

    O2, Kin2, Kout2 = _LAYERS[2]
    Wm, bm, WeR2, beR2, attR2, Wsf, bsf = Ws[2]
    xmtab2, xself2, anmax2 = _tc_finalize_project(O1, Kout1, O2, Kin2)(
        P1, xself1, WeR1, beR1, Wm, bm, attR2, Wsf, bsf)
    consts2 = _head_consts(WeR2, beR2, attR2, anmax2, attmm)
    P2 = _make_sc_edge(O2, Kin2, Kout2)(
        src, dst, attrw, consts2, xmtab2, zeros[2])

    O3, Kin3, Kout3 = _LAYERS[3]
    Wm, bm, WeR3, beR3, attR3, Wsf, bsf = Ws[3]
    xmtab3, xself3, anmax3 = _tc_finalize_project(O2, Kout2, O3, Kin3)(
        P2, xself2, WeR2, beR2, Wm, bm, attR3, Wsf, bsf)
    consts3 = _head_consts(WeR3, beR3, attR3, anmax3, attmm)
    P3 = _make_sc_edge(O3, Kin3, Kout3)(
        src, dst, attrw, consts3, xmtab3, zeros[3])

    return _tc_final_pool(O3, Kout3)(
        P3, xself3, WeR3, beR3, batch3d, Wout, bout.reshape(1, 2))
